# Initial kernel scaffold; baseline (speedup 1.0000x reference)
import jax
import jax.numpy as jnp
from jax.experimental import pallas as pl


def kernel(x, he_index, he_attr, he_mark, he_count, n2e_W1, n2e_b1, n2e_g1, n2e_be1, n2e_W2, n2e_b2, e2n_W1, e2n_b1, e2n_g1, e2n_be1, e2n_W2, e2n_b2, ln_g, ln_b):
    raise NotImplementedError("write your pallas kernel here")


# speedup vs baseline: 3.4582x; 3.4582x over previous
import functools

import jax
import jax.numpy as jnp
from jax import lax
from jax.experimental import pallas as pl
from jax.experimental.pallas import tpu as pltpu
from jax.experimental.pallas import tpu_sc as plsc

F32 = jnp.float32


def _tc1_body(x_ref, w_ref, o_ref):
    o_ref[...] = jnp.dot(x_ref[...], w_ref[...], preferred_element_type=F32)


def _layernorm_rows(h, g, b):
    mu = jnp.mean(h, axis=1, keepdims=True)
    var = jnp.mean((h - mu) * (h - mu), axis=1, keepdims=True)
    return (h - mu) / jnp.sqrt(var + 1e-5) * g + b


def _tc2_body(g1_ref, m16_ref, w1b_ref, b1_ref, g_ref, be_ref, w2_ref, b2_ref,
              o_ref):
    h = g1_ref[...] + jnp.dot(m16_ref[...], w1b_ref[...],
                              preferred_element_type=F32) + b1_ref[...]
    a = jnp.maximum(_layernorm_rows(h, g_ref[...], be_ref[...]), 0.0)
    msg = jnp.dot(a, w2_ref[...], preferred_element_type=F32) + b2_ref[...]
    half = msg.shape[1] // 2
    o_ref[0, ...] = msg[:, :half]
    o_ref[1, ...] = msg[:, half:]


def _tc3_body(agg_ref, attr_ref, cnt_ref, v1a_ref, v1b_ref, b1_ref, g_ref,
              be_ref, v2_ref, b2_ref, o_ref):
    agg = jnp.concatenate([agg_ref[0, ...], agg_ref[1, ...]], axis=1)
    agg = agg / (cnt_ref[...] + 1e-6)
    h = (jnp.dot(attr_ref[...], v1a_ref[...], preferred_element_type=F32)
         + jnp.dot(agg, v1b_ref[...], preferred_element_type=F32)
         + b1_ref[...])
    a = jnp.maximum(_layernorm_rows(h, g_ref[...], be_ref[...]), 0.0)
    mh = jnp.maximum(
        jnp.dot(a, v2_ref[...], preferred_element_type=F32) + b2_ref[...], 0.0)
    half = mh.shape[1] // 2
    o_ref[0, ...] = mh[:, :half]
    o_ref[1, ...] = mh[:, half:]


def _tc4_body(o2_ref, deg_ref, x_ref, g_ref, b_ref, y_ref):
    o = jnp.concatenate([o2_ref[0, ...], o2_ref[1, ...]], axis=1)
    deg = jnp.sum(deg_ref[...], axis=1)[:, None]
    o = o / (deg + 1e-6)
    y_ref[...] = x_ref[...] + _layernorm_rows(o, g_ref[...], b_ref[...])


_MESH = dict(core_axis_name="c", subcore_axis_name="s")


def _sc_gather_body(NCHUNK, xw_hbm, mc0_hbm, mc1_hbm, nidx2d_hbm,
                    hidx2d_hbm, zm_hbm, g1_hbm, m16_hbm,
                    nib, hib, xbuf, mbuf, mark0_v, mark1_v, sem1):
    c = lax.axis_index("c")
    s = lax.axis_index("s")
    wid = s * 2 + c
    pltpu.sync_copy(mc0_hbm, mark0_v)
    pltpu.sync_copy(mc1_hbm, mark1_v)
    pltpu.sync_copy(zm_hbm, mbuf)
    iota = lax.iota(jnp.int32, 16)
    col0 = jnp.zeros((16,), jnp.int32)
    col1 = jnp.ones((16,), jnp.int32)

    def body(t, carry):
        tt = wid + t * 32

        @pl.when(tt < NCHUNK)
        def _():
            pltpu.sync_copy(nidx2d_hbm.at[tt], nib.at[0])
            pltpu.sync_copy(hidx2d_hbm.at[tt], hib.at[0])
            pltpu.async_copy(xw_hbm.at[nib.at[0]], xbuf, sem1).wait()
            for j in range(8):
                hi = hib[0, pl.ds(j * 16, 16)]
                v0 = plsc.load_gather(mark0_v, [hi])
                v1 = plsc.load_gather(mark1_v, [hi])
                rows = iota + (j * 16)
                plsc.store_scatter(mbuf, [rows, col0], v0)
                plsc.store_scatter(mbuf, [rows, col1], v1)
            pltpu.sync_copy(xbuf, g1_hbm.at[pl.ds(tt * 128, 128)])
            pltpu.sync_copy(mbuf, m16_hbm.at[pl.ds(tt * 128, 128)])

        return carry

    lax.fori_loop(0, (NCHUNK + 31) // 32, body, 0)


def _sc_scatter_he_body(NCHUNK, SPR,
                        msg_hbm, hidx2d_hbm, zeros_hbm, agg_hbm,
                        idxbuf, rowbuf, agg_sp):
    c = lax.axis_index("c")
    s = lax.axis_index("s")
    pltpu.sync_copy(zeros_hbm.at[pl.ds(s * SPR, SPR)],
                    agg_sp.at[pl.ds(s * SPR, SPR)])
    plsc.subcore_barrier()

    def body(t, carry):
        tt = s + t * 16

        @pl.when(tt < NCHUNK)
        def _():
            pltpu.sync_copy(hidx2d_hbm.at[tt], idxbuf.at[0])
            pltpu.sync_copy(msg_hbm.at[c].at[pl.ds(tt * 128, 128)], rowbuf)
            pltpu.sync_copy(rowbuf, agg_sp.at[idxbuf.at[0]], add=True)

        return carry

    lax.fori_loop(0, (NCHUNK + 15) // 16, body, 0)
    plsc.subcore_barrier()
    pltpu.sync_copy(agg_sp.at[pl.ds(s * SPR, SPR)],
                    agg_hbm.at[c].at[pl.ds(s * SPR, SPR)])


def _sc_scatter_node_body(NCHUNK, SPR,
                          mh_hbm, hidx2d_hbm, nidx2d_hbm, zeros_hbm, z1d_hbm,
                          out_hbm, deg_hbm,
                          hib, nib, rowbuf, cnt_v, out_sp, sem):
    c = lax.axis_index("c")
    s = lax.axis_index("s")
    pltpu.sync_copy(zeros_hbm.at[pl.ds(s * SPR, SPR)],
                    out_sp.at[pl.ds(s * SPR, SPR)])
    pltpu.sync_copy(z1d_hbm, cnt_v)
    plsc.subcore_barrier()
    ones_f = jnp.ones((16,), F32)

    def body(t, carry):
        tt = s + t * 16

        @pl.when(tt < NCHUNK)
        def _():
            pltpu.sync_copy(hidx2d_hbm.at[tt], hib.at[0])
            pltpu.sync_copy(nidx2d_hbm.at[tt], nib.at[0])
            pltpu.async_copy(mh_hbm.at[c].at[hib.at[0]], rowbuf, sem).wait()
            pltpu.sync_copy(rowbuf, out_sp.at[nib.at[0]], add=True)
            for j in range(8):
                ni = nib[0, pl.ds(j * 16, 16)]
                plsc.addupdate_scatter(cnt_v, [ni], ones_f)

        return carry

    lax.fori_loop(0, (NCHUNK + 15) // 16, body, 0)
    plsc.subcore_barrier()
    pltpu.sync_copy(out_sp.at[pl.ds(s * SPR, SPR)],
                    out_hbm.at[c].at[pl.ds(s * SPR, SPR)])

    @pl.when(c == 0)
    def _():
        pltpu.sync_copy(cnt_v, deg_hbm.at[s])


def kernel(x, he_index, he_attr, he_mark, he_count,
           n2e_W1, n2e_b1, n2e_g1, n2e_be1, n2e_W2, n2e_b2,
           e2n_W1, e2n_b1, e2n_g1, e2n_be1, e2n_W2, e2n_b2,
           ln_g, ln_b):
    N, D = x.shape
    H, HE = he_attr.shape
    E = he_index.shape[1]
    HID = n2e_W1.shape[1]
    DH = D // 2
    NCHUNK = E // 128
    HP = 10240
    SPR = HP // 16

    node_ids = he_index[0]
    he_ids = he_index[1]
    hidx2d = he_ids.reshape(NCHUNK, 128)
    nidx2d = node_ids.reshape(NCHUNK, 128)

    mark_c0 = he_mark[:, 0]
    mark_c1 = he_mark[:, 1]
    W1b16 = jnp.pad(n2e_W1[D:], ((0, 16 - (n2e_W1.shape[0] - D)), (0, 0)))
    W1a = n2e_W1[:D]
    V1a = e2n_W1[:HE]
    V1b = e2n_W1[HE:]
    b1 = n2e_b1.reshape(1, HID)
    g1 = n2e_g1.reshape(1, HID)
    be1 = n2e_be1.reshape(1, HID)
    b2 = n2e_b2.reshape(1, HID)
    eb1 = e2n_b1.reshape(1, HID)
    eg1 = e2n_g1.reshape(1, HID)
    ebe1 = e2n_be1.reshape(1, HID)
    eb2 = e2n_b2.reshape(1, D)
    cnt = he_count.reshape(H, 1)
    lng = ln_g.reshape(1, D)
    lnb = ln_b.reshape(1, D)

    zeros_half = jnp.zeros((HP, DH), F32)
    zeros_1d = jnp.zeros((HP,), F32)
    zeros_m = jnp.zeros((128, 16), F32)

    BN = 2000
    xW = pl.pallas_call(
        _tc1_body,
        grid=(N // BN,),
        in_specs=[pl.BlockSpec((BN, D), lambda i: (i, 0)),
                  pl.BlockSpec((D, HID), lambda i: (0, 0))],
        out_specs=pl.BlockSpec((BN, HID), lambda i: (i, 0)),
        out_shape=jax.ShapeDtypeStruct((N, HID), F32),
    )(x, W1a)

    mesh = plsc.VectorSubcoreMesh(**_MESH)
    sc1 = functools.partial(
        pl.kernel,
        out_type=[jax.ShapeDtypeStruct((E, HID), F32),
                  jax.ShapeDtypeStruct((E, 16), F32)],
        mesh=mesh,
        compiler_params=pltpu.CompilerParams(needs_layout_passes=False),
        scratch_types=[
            pltpu.VMEM((1, 128), jnp.int32),
            pltpu.VMEM((1, 128), jnp.int32),
            pltpu.VMEM((128, HID), F32),
            pltpu.VMEM((128, 16), F32),
            pltpu.VMEM((H,), F32),
            pltpu.VMEM((H,), F32),
            pltpu.SemaphoreType.DMA,
        ],
    )(functools.partial(_sc_gather_body, NCHUNK))
    G1, M16 = sc1(xW, mark_c0, mark_c1, nidx2d, hidx2d, zeros_m)

    BE = 1600
    msg2 = pl.pallas_call(
        _tc2_body,
        grid=(E // BE,),
        in_specs=[pl.BlockSpec((BE, HID), lambda i: (i, 0)),
                  pl.BlockSpec((BE, 16), lambda i: (i, 0)),
                  pl.BlockSpec((16, HID), lambda i: (0, 0)),
                  pl.BlockSpec((1, HID), lambda i: (0, 0)),
                  pl.BlockSpec((1, HID), lambda i: (0, 0)),
                  pl.BlockSpec((1, HID), lambda i: (0, 0)),
                  pl.BlockSpec((HID, HID), lambda i: (0, 0)),
                  pl.BlockSpec((1, HID), lambda i: (0, 0))],
        out_specs=pl.BlockSpec((2, BE, DH), lambda i: (0, i, 0)),
        out_shape=jax.ShapeDtypeStruct((2, E, DH), F32),
    )(G1, M16, W1b16, b1, g1, be1, n2e_W2, b2)

    sc2 = functools.partial(
        pl.kernel,
        out_type=jax.ShapeDtypeStruct((2, HP, DH), F32),
        mesh=mesh,
        compiler_params=pltpu.CompilerParams(needs_layout_passes=False),
        scratch_types=[
            pltpu.VMEM((1, 128), jnp.int32),
            pltpu.VMEM((128, DH), F32),
            pltpu.VMEM_SHARED((HP, DH), F32),
        ],
    )(functools.partial(_sc_scatter_he_body, NCHUNK, SPR))
    agg2 = sc2(msg2, hidx2d, zeros_half)

    BH = 2000
    mh2 = pl.pallas_call(
        _tc3_body,
        grid=(H // BH,),
        in_specs=[pl.BlockSpec((2, BH, DH), lambda i: (0, i, 0)),
                  pl.BlockSpec((BH, HE), lambda i: (i, 0)),
                  pl.BlockSpec((BH, 1), lambda i: (i, 0)),
                  pl.BlockSpec((HE, HID), lambda i: (0, 0)),
                  pl.BlockSpec((HID, HID), lambda i: (0, 0)),
                  pl.BlockSpec((1, HID), lambda i: (0, 0)),
                  pl.BlockSpec((1, HID), lambda i: (0, 0)),
                  pl.BlockSpec((1, HID), lambda i: (0, 0)),
                  pl.BlockSpec((HID, D), lambda i: (0, 0)),
                  pl.BlockSpec((1, D), lambda i: (0, 0))],
        out_specs=pl.BlockSpec((2, BH, DH), lambda i: (0, i, 0)),
        out_shape=jax.ShapeDtypeStruct((2, HP, DH), F32),
    )(agg2, he_attr, cnt, V1a, V1b, eb1, eg1, ebe1, e2n_W2, eb2)

    sc3 = functools.partial(
        pl.kernel,
        out_type=[jax.ShapeDtypeStruct((2, HP, DH), F32),
                  jax.ShapeDtypeStruct((16, HP), F32)],
        mesh=mesh,
        compiler_params=pltpu.CompilerParams(needs_layout_passes=False),
        scratch_types=[
            pltpu.VMEM((1, 128), jnp.int32),
            pltpu.VMEM((1, 128), jnp.int32),
            pltpu.VMEM((128, DH), F32),
            pltpu.VMEM((HP,), F32),
            pltpu.VMEM_SHARED((HP, DH), F32),
            pltpu.SemaphoreType.DMA,
        ],
    )(functools.partial(_sc_scatter_node_body, NCHUNK, SPR))
    out2, deg16 = sc3(mh2, hidx2d, nidx2d, zeros_half, zeros_1d)
    deg16 = deg16.T

    y = pl.pallas_call(
        _tc4_body,
        grid=(N // BN,),
        in_specs=[pl.BlockSpec((2, BN, DH), lambda i: (0, i, 0)),
                  pl.BlockSpec((BN, 16), lambda i: (i, 0)),
                  pl.BlockSpec((BN, D), lambda i: (i, 0)),
                  pl.BlockSpec((1, D), lambda i: (0, 0)),
                  pl.BlockSpec((1, D), lambda i: (0, 0))],
        out_specs=pl.BlockSpec((BN, D), lambda i: (i, 0)),
        out_shape=jax.ShapeDtypeStruct((N, D), F32),
    )(out2, deg16, x, lng, lnb)
    return y


# speedup vs baseline: 5.0330x; 1.4554x over previous
import functools

import jax
import jax.numpy as jnp
from jax import lax
from jax.experimental import pallas as pl
from jax.experimental.pallas import tpu as pltpu
from jax.experimental.pallas import tpu_sc as plsc

F32 = jnp.float32


def _tc1_body(x_ref, w_ref, o_ref):
    o_ref[...] = jnp.dot(x_ref[...], w_ref[...], preferred_element_type=F32)


def _layernorm_rows(h, g, b):
    mu = jnp.mean(h, axis=1, keepdims=True)
    var = jnp.mean((h - mu) * (h - mu), axis=1, keepdims=True)
    return (h - mu) / jnp.sqrt(var + 1e-5) * g + b


def _tc2_body(g1_ref, m16_ref, w1b_ref, b1_ref, g_ref, be_ref, w2_ref, b2_ref,
              o_ref):
    h = g1_ref[...] + jnp.dot(m16_ref[...], w1b_ref[...],
                              preferred_element_type=F32) + b1_ref[...]
    a = jnp.maximum(_layernorm_rows(h, g_ref[...], be_ref[...]), 0.0)
    msg = jnp.dot(a, w2_ref[...], preferred_element_type=F32) + b2_ref[...]
    half = msg.shape[1] // 2
    o_ref[0, ...] = msg[:, :half]
    o_ref[1, ...] = msg[:, half:]


def _tc3_body(agg_ref, attr_ref, cnt_ref, v1a_ref, v1b_ref, b1_ref, g_ref,
              be_ref, v2_ref, b2_ref, o_ref):
    agg = jnp.concatenate([agg_ref[0, ...], agg_ref[1, ...]], axis=1)
    agg = agg / (cnt_ref[...] + 1e-6)
    h = (jnp.dot(attr_ref[...], v1a_ref[...], preferred_element_type=F32)
         + jnp.dot(agg, v1b_ref[...], preferred_element_type=F32)
         + b1_ref[...])
    a = jnp.maximum(_layernorm_rows(h, g_ref[...], be_ref[...]), 0.0)
    mh = jnp.maximum(
        jnp.dot(a, v2_ref[...], preferred_element_type=F32) + b2_ref[...], 0.0)
    half = mh.shape[1] // 2
    o_ref[0, ...] = mh[:, :half]
    o_ref[1, ...] = mh[:, half:]


def _tc4_body(o2_ref, deg_ref, x_ref, g_ref, b_ref, y_ref):
    o = jnp.concatenate([o2_ref[0, ...], o2_ref[1, ...]], axis=1)
    deg = jnp.sum(deg_ref[...], axis=1)[:, None]
    o = o / (deg + 1e-6)
    y_ref[...] = x_ref[...] + _layernorm_rows(o, g_ref[...], b_ref[...])


_MESH = dict(core_axis_name="c", subcore_axis_name="s")


def _sc_gather_body(NCHUNK, CPT, xw_hbm, mc0_hbm, mc1_hbm, nidx2d_hbm,
                    hidx2d_hbm, zm_hbm, z1d_hbm, g1_hbm, m16_hbm, deg_hbm,
                    idxn, idxh, xbuf0, xbuf1, mbuf, mark0_v, mark1_v, cnt_v,
                    sem0, sem1):
    c = lax.axis_index("c")
    s = lax.axis_index("s")
    wid = s * 2 + c
    base = wid * CPT
    pltpu.sync_copy(nidx2d_hbm.at[pl.ds(base, CPT)], idxn)
    pltpu.sync_copy(hidx2d_hbm.at[pl.ds(base, CPT)], idxh)
    pltpu.sync_copy(mc0_hbm, mark0_v)
    pltpu.sync_copy(mc1_hbm, mark1_v)
    pltpu.sync_copy(zm_hbm, mbuf)
    pltpu.sync_copy(z1d_hbm, cnt_v)
    iota = lax.iota(jnp.int32, 16)
    col0 = jnp.zeros((16,), jnp.int32)
    col1 = jnp.ones((16,), jnp.int32)
    ones_f = jnp.ones((16,), F32)

    @pl.when(base < NCHUNK)
    def _():
        pltpu.async_copy(xw_hbm.at[idxn.at[0]], xbuf0, sem0)

    def emit(i, tt, xbuf, sem):
        for j in range(8):
            hi = idxh[i, pl.ds(j * 16, 16)]
            v0 = plsc.load_gather(mark0_v, [hi])
            v1 = plsc.load_gather(mark1_v, [hi])
            rows = iota + (j * 16)
            plsc.store_scatter(mbuf, [rows, col0], v0)
            plsc.store_scatter(mbuf, [rows, col1], v1)
            ni = idxn[i, pl.ds(j * 16, 16)]
            plsc.addupdate_scatter(cnt_v, [ni], ones_f)
        pltpu.make_async_copy(xw_hbm.at[idxn.at[i]], xbuf, sem).wait()
        pltpu.sync_copy(xbuf, g1_hbm.at[pl.ds(tt * 128, 128)])
        pltpu.sync_copy(mbuf, m16_hbm.at[pl.ds(tt * 128, 128)])

    def body(g, carry):
        i0 = 2 * g
        i1 = i0 + 1
        t0 = base + i0
        t1 = base + i1

        @pl.when(t1 < NCHUNK)
        def _():
            pltpu.async_copy(xw_hbm.at[idxn.at[i1]], xbuf1, sem1)

        @pl.when(t0 < NCHUNK)
        def _():
            emit(i0, t0, xbuf0, sem0)

        @pl.when(((i0 + 2) < CPT) & ((t0 + 2) < NCHUNK))
        def _():
            pltpu.async_copy(xw_hbm.at[idxn.at[i0 + 2]], xbuf0, sem0)

        @pl.when(t1 < NCHUNK)
        def _():
            emit(i1, t1, xbuf1, sem1)

        return carry

    lax.fori_loop(0, CPT // 2, body, 0)
    pltpu.sync_copy(cnt_v, deg_hbm.at[wid])


def _sc_scatter_he_body(NCHUNK, CPT, SPR,
                        msg_hbm, hidx2d_hbm, zeros_hbm, agg_hbm,
                        idxh, rowbuf0, rowbuf1, agg_sp, sem0, sem1):
    c = lax.axis_index("c")
    s = lax.axis_index("s")
    base = s * CPT
    pltpu.sync_copy(zeros_hbm.at[pl.ds(s * SPR, SPR)],
                    agg_sp.at[pl.ds(s * SPR, SPR)])
    plsc.subcore_barrier()

    @pl.when(base < NCHUNK)
    def _():
        pltpu.async_copy(msg_hbm.at[c].at[pl.ds(base * 128, 128)], rowbuf0,
                         sem0)

    def emit(i, tt, rowbuf, sem):
        pltpu.make_async_copy(msg_hbm.at[c].at[pl.ds(tt * 128, 128)], rowbuf,
                              sem).wait()
        pltpu.sync_copy(rowbuf, agg_sp.at[idxh.at[i]], add=True)

    def body(g, carry):
        i0 = 2 * g
        i1 = i0 + 1
        t0 = base + i0
        t1 = base + i1

        @pl.when(t1 < NCHUNK)
        def _():
            pltpu.async_copy(msg_hbm.at[c].at[pl.ds(t1 * 128, 128)], rowbuf1,
                             sem1)

        pltpu.sync_copy(hidx2d_hbm.at[pl.ds(t0, 2)], idxh)

        @pl.when(t0 < NCHUNK)
        def _():
            emit(0, t0, rowbuf0, sem0)

        @pl.when(((i0 + 2) < CPT) & ((t0 + 2) < NCHUNK))
        def _():
            pltpu.async_copy(msg_hbm.at[c].at[pl.ds((t0 + 2) * 128, 128)],
                             rowbuf0, sem0)

        @pl.when(t1 < NCHUNK)
        def _():
            emit(1, t1, rowbuf1, sem1)

        return carry

    lax.fori_loop(0, CPT // 2, body, 0)
    plsc.subcore_barrier()
    pltpu.sync_copy(agg_sp.at[pl.ds(s * SPR, SPR)],
                    agg_hbm.at[c].at[pl.ds(s * SPR, SPR)])


def _sc_scatter_node_body(NCHUNK, CPT, SPR,
                          mh_hbm, hidx2d_hbm, nidx2d_hbm, zeros_hbm,
                          out_hbm,
                          idxh4, idxn4, rowbuf0, rowbuf1, out_sp,
                          sem0, sem1):
    c = lax.axis_index("c")
    s = lax.axis_index("s")
    base = s * CPT
    pltpu.sync_copy(zeros_hbm.at[pl.ds(s * SPR, SPR)],
                    out_sp.at[pl.ds(s * SPR, SPR)])
    plsc.subcore_barrier()
    pltpu.sync_copy(hidx2d_hbm.at[pl.ds(base, 2)], idxh4.at[pl.ds(0, 2)])
    pltpu.sync_copy(nidx2d_hbm.at[pl.ds(base, 2)], idxn4.at[pl.ds(0, 2)])

    @pl.when(base < NCHUNK)
    def _():
        pltpu.async_copy(mh_hbm.at[c].at[idxh4.at[0]], rowbuf0, sem0)

    def body(g, carry):
        i0 = 2 * g
        t0 = base + i0
        t1 = t0 + 1
        bank = (g % 2) * 2
        nbank = 2 - bank

        @pl.when(t1 < NCHUNK)
        def _():
            pltpu.async_copy(mh_hbm.at[c].at[idxh4.at[bank + 1]], rowbuf1,
                             sem1)

        pltpu.sync_copy(hidx2d_hbm.at[pl.ds(t0 + 2, 2)],
                        idxh4.at[pl.ds(nbank, 2)])
        pltpu.sync_copy(nidx2d_hbm.at[pl.ds(t0 + 2, 2)],
                        idxn4.at[pl.ds(nbank, 2)])

        @pl.when(t0 < NCHUNK)
        def _():
            pltpu.make_async_copy(mh_hbm.at[c].at[idxh4.at[bank]], rowbuf0,
                                  sem0).wait()
            pltpu.sync_copy(rowbuf0, out_sp.at[idxn4.at[bank]], add=True)

        @pl.when(((i0 + 2) < CPT) & ((t0 + 2) < NCHUNK))
        def _():
            pltpu.async_copy(mh_hbm.at[c].at[idxh4.at[nbank]], rowbuf0, sem0)

        @pl.when(t1 < NCHUNK)
        def _():
            pltpu.make_async_copy(mh_hbm.at[c].at[idxh4.at[bank + 1]],
                                  rowbuf1, sem1).wait()
            pltpu.sync_copy(rowbuf1, out_sp.at[idxn4.at[bank + 1]], add=True)

        return carry

    lax.fori_loop(0, CPT // 2, body, 0)
    plsc.subcore_barrier()
    pltpu.sync_copy(out_sp.at[pl.ds(s * SPR, SPR)],
                    out_hbm.at[c].at[pl.ds(s * SPR, SPR)])


def kernel(x, he_index, he_attr, he_mark, he_count,
           n2e_W1, n2e_b1, n2e_g1, n2e_be1, n2e_W2, n2e_b2,
           e2n_W1, e2n_b1, e2n_g1, e2n_be1, e2n_W2, e2n_b2,
           ln_g, ln_b):
    N, D = x.shape
    H, HE = he_attr.shape
    E = he_index.shape[1]
    HID = n2e_W1.shape[1]
    DH = D // 2
    NCHUNK = E // 128
    HP = 10240
    SPR = HP // 16

    node_ids = he_index[0]
    he_ids = he_index[1]
    NCP = 1280
    CPT1 = NCP // 32
    CPT2 = NCP // 16
    hidx2d = jnp.pad(he_ids.reshape(NCHUNK, 128),
                     ((0, NCP + 32 - NCHUNK), (0, 0)))
    nidx2d = jnp.pad(node_ids.reshape(NCHUNK, 128),
                     ((0, NCP + 32 - NCHUNK), (0, 0)))

    mark_c0 = he_mark[:, 0]
    mark_c1 = he_mark[:, 1]
    W1b16 = jnp.pad(n2e_W1[D:], ((0, 16 - (n2e_W1.shape[0] - D)), (0, 0)))
    W1a = n2e_W1[:D]
    V1a = e2n_W1[:HE]
    V1b = e2n_W1[HE:]
    b1 = n2e_b1.reshape(1, HID)
    g1 = n2e_g1.reshape(1, HID)
    be1 = n2e_be1.reshape(1, HID)
    b2 = n2e_b2.reshape(1, HID)
    eb1 = e2n_b1.reshape(1, HID)
    eg1 = e2n_g1.reshape(1, HID)
    ebe1 = e2n_be1.reshape(1, HID)
    eb2 = e2n_b2.reshape(1, D)
    cnt = he_count.reshape(H, 1)
    lng = ln_g.reshape(1, D)
    lnb = ln_b.reshape(1, D)

    zeros_half = jnp.zeros((HP, DH), F32)
    zeros_1d = jnp.zeros((HP,), F32)
    zeros_m = jnp.zeros((128, 16), F32)

    BN = 2000
    xW = pl.pallas_call(
        _tc1_body,
        grid=(N // BN,),
        in_specs=[pl.BlockSpec((BN, D), lambda i: (i, 0)),
                  pl.BlockSpec((D, HID), lambda i: (0, 0))],
        out_specs=pl.BlockSpec((BN, HID), lambda i: (i, 0)),
        out_shape=jax.ShapeDtypeStruct((N, HID), F32),
    )(x, W1a)

    mesh = plsc.VectorSubcoreMesh(**_MESH)
    sc1 = functools.partial(
        pl.kernel,
        out_type=[jax.ShapeDtypeStruct((E, HID), F32),
                  jax.ShapeDtypeStruct((E, 16), F32),
                  jax.ShapeDtypeStruct((32, HP), F32)],
        mesh=mesh,
        compiler_params=pltpu.CompilerParams(needs_layout_passes=False),
        scratch_types=[
            pltpu.VMEM((CPT1, 128), jnp.int32),
            pltpu.VMEM((CPT1, 128), jnp.int32),
            pltpu.VMEM((128, HID), F32),
            pltpu.VMEM((128, HID), F32),
            pltpu.VMEM((128, 16), F32),
            pltpu.VMEM((H,), F32),
            pltpu.VMEM((H,), F32),
            pltpu.VMEM((HP,), F32),
            pltpu.SemaphoreType.DMA,
            pltpu.SemaphoreType.DMA,
        ],
    )(functools.partial(_sc_gather_body, NCHUNK, CPT1))
    G1, M16, deg32 = sc1(xW, mark_c0, mark_c1, nidx2d, hidx2d, zeros_m,
                         zeros_1d)
    deg32 = deg32.T

    BE = 1600
    msg2 = pl.pallas_call(
        _tc2_body,
        grid=(E // BE,),
        in_specs=[pl.BlockSpec((BE, HID), lambda i: (i, 0)),
                  pl.BlockSpec((BE, 16), lambda i: (i, 0)),
                  pl.BlockSpec((16, HID), lambda i: (0, 0)),
                  pl.BlockSpec((1, HID), lambda i: (0, 0)),
                  pl.BlockSpec((1, HID), lambda i: (0, 0)),
                  pl.BlockSpec((1, HID), lambda i: (0, 0)),
                  pl.BlockSpec((HID, HID), lambda i: (0, 0)),
                  pl.BlockSpec((1, HID), lambda i: (0, 0))],
        out_specs=pl.BlockSpec((2, BE, DH), lambda i: (0, i, 0)),
        out_shape=jax.ShapeDtypeStruct((2, E, DH), F32),
    )(G1, M16, W1b16, b1, g1, be1, n2e_W2, b2)

    sc2 = functools.partial(
        pl.kernel,
        out_type=jax.ShapeDtypeStruct((2, HP, DH), F32),
        mesh=mesh,
        compiler_params=pltpu.CompilerParams(needs_layout_passes=False),
        scratch_types=[
            pltpu.VMEM((2, 128), jnp.int32),
            pltpu.VMEM((128, DH), F32),
            pltpu.VMEM((128, DH), F32),
            pltpu.VMEM_SHARED((HP, DH), F32),
            pltpu.SemaphoreType.DMA,
            pltpu.SemaphoreType.DMA,
        ],
    )(functools.partial(_sc_scatter_he_body, NCHUNK, CPT2, SPR))
    agg2 = sc2(msg2, hidx2d, zeros_half)

    BH = 2000
    mh2 = pl.pallas_call(
        _tc3_body,
        grid=(H // BH,),
        in_specs=[pl.BlockSpec((2, BH, DH), lambda i: (0, i, 0)),
                  pl.BlockSpec((BH, HE), lambda i: (i, 0)),
                  pl.BlockSpec((BH, 1), lambda i: (i, 0)),
                  pl.BlockSpec((HE, HID), lambda i: (0, 0)),
                  pl.BlockSpec((HID, HID), lambda i: (0, 0)),
                  pl.BlockSpec((1, HID), lambda i: (0, 0)),
                  pl.BlockSpec((1, HID), lambda i: (0, 0)),
                  pl.BlockSpec((1, HID), lambda i: (0, 0)),
                  pl.BlockSpec((HID, D), lambda i: (0, 0)),
                  pl.BlockSpec((1, D), lambda i: (0, 0))],
        out_specs=pl.BlockSpec((2, BH, DH), lambda i: (0, i, 0)),
        out_shape=jax.ShapeDtypeStruct((2, HP, DH), F32),
    )(agg2, he_attr, cnt, V1a, V1b, eb1, eg1, ebe1, e2n_W2, eb2)

    sc3 = functools.partial(
        pl.kernel,
        out_type=jax.ShapeDtypeStruct((2, HP, DH), F32),
        mesh=mesh,
        compiler_params=pltpu.CompilerParams(needs_layout_passes=False),
        scratch_types=[
            pltpu.VMEM((4, 128), jnp.int32),
            pltpu.VMEM((4, 128), jnp.int32),
            pltpu.VMEM((128, DH), F32),
            pltpu.VMEM((128, DH), F32),
            pltpu.VMEM_SHARED((HP, DH), F32),
            pltpu.SemaphoreType.DMA,
            pltpu.SemaphoreType.DMA,
        ],
    )(functools.partial(_sc_scatter_node_body, NCHUNK, CPT2, SPR))
    out2 = sc3(mh2, hidx2d, nidx2d, zeros_half)

    y = pl.pallas_call(
        _tc4_body,
        grid=(N // BN,),
        in_specs=[pl.BlockSpec((2, BN, DH), lambda i: (0, i, 0)),
                  pl.BlockSpec((BN, 32), lambda i: (i, 0)),
                  pl.BlockSpec((BN, D), lambda i: (i, 0)),
                  pl.BlockSpec((1, D), lambda i: (0, 0)),
                  pl.BlockSpec((1, D), lambda i: (0, 0))],
        out_specs=pl.BlockSpec((BN, D), lambda i: (i, 0)),
        out_shape=jax.ShapeDtypeStruct((N, D), F32),
    )(out2, deg32, x, lng, lnb)
    return y


# speedup vs baseline: 5.2756x; 1.0482x over previous
import functools

import jax
import jax.numpy as jnp
from jax import lax
from jax.experimental import pallas as pl
from jax.experimental.pallas import tpu as pltpu
from jax.experimental.pallas import tpu_sc as plsc

F32 = jnp.float32


def _tc1_body(x_ref, w_ref, o_ref):
    o_ref[...] = jnp.dot(x_ref[...], w_ref[...], preferred_element_type=F32)


def _layernorm_rows(h, g, b):
    mu = jnp.mean(h, axis=1, keepdims=True)
    var = jnp.mean((h - mu) * (h - mu), axis=1, keepdims=True)
    return (h - mu) / jnp.sqrt(var + 1e-5) * g + b


def _tc2_body(g1_ref, m16_ref, w1b_ref, b1_ref, g_ref, be_ref, w2_ref, b2_ref,
              o_ref):
    h = (g1_ref[...]
         + jnp.dot(m16_ref[...], w1b_ref[...], preferred_element_type=F32)
         + b1_ref[...])
    a = jnp.maximum(_layernorm_rows(h, g_ref[...], be_ref[...]), 0.0)
    msg = jnp.dot(a.astype(jnp.bfloat16), w2_ref[...].astype(jnp.bfloat16),
                  preferred_element_type=F32) + b2_ref[...]
    half = msg.shape[1] // 2
    o_ref[0, ...] = msg[:, :half]
    o_ref[1, ...] = msg[:, half:]


def _tc3_body(agg_ref, attr_ref, cnt_ref, v1a_ref, v1b_ref, b1_ref, g_ref,
              be_ref, v2_ref, b2_ref, o_ref):
    agg = jnp.concatenate([agg_ref[0, ...], agg_ref[1, ...]], axis=1)
    agg = agg / (cnt_ref[...] + 1e-6)
    h = (jnp.dot(attr_ref[...], v1a_ref[...], preferred_element_type=F32)
         + jnp.dot(agg.astype(jnp.bfloat16), v1b_ref[...].astype(jnp.bfloat16),
                   preferred_element_type=F32)
         + b1_ref[...])
    a = jnp.maximum(_layernorm_rows(h, g_ref[...], be_ref[...]), 0.0)
    mh = jnp.maximum(
        jnp.dot(a.astype(jnp.bfloat16), v2_ref[...].astype(jnp.bfloat16),
                preferred_element_type=F32) + b2_ref[...], 0.0)
    half = mh.shape[1] // 2
    o_ref[0, ...] = mh[:, :half]
    o_ref[1, ...] = mh[:, half:]


def _tc4_body(o2_ref, deg_ref, x_ref, g_ref, b_ref, y_ref):
    o = jnp.concatenate([o2_ref[0, ...], o2_ref[1, ...]], axis=1)
    deg = jnp.sum(deg_ref[...], axis=1)[:, None]
    o = o / (deg + 1e-6)
    y_ref[...] = x_ref[...] + _layernorm_rows(o, g_ref[...], b_ref[...])


_MESH = dict(core_axis_name="c", subcore_axis_name="s")


def _sc_gather_body(NCHUNK, CPT, xw_hbm, mc0_hbm, mc1_hbm, nidx2d_hbm,
                    hidx2d_hbm, zm_hbm, z1d_hbm, g1_hbm, m16_hbm, deg_hbm,
                    idxn, idxh, xbuf0, xbuf1, mbuf, mark0_v, mark1_v, cnt_v,
                    sem0, sem1):
    c = lax.axis_index("c")
    s = lax.axis_index("s")
    wid = s * 2 + c
    base = wid * CPT
    pltpu.sync_copy(nidx2d_hbm.at[pl.ds(base, CPT)], idxn)
    pltpu.sync_copy(hidx2d_hbm.at[pl.ds(base, CPT)], idxh)
    pltpu.sync_copy(mc0_hbm, mark0_v)
    pltpu.sync_copy(mc1_hbm, mark1_v)
    pltpu.sync_copy(zm_hbm, mbuf)
    pltpu.sync_copy(z1d_hbm, cnt_v)
    iota = lax.iota(jnp.int32, 16)
    col0 = jnp.zeros((16,), jnp.int32)
    col1 = jnp.ones((16,), jnp.int32)
    ones_f = jnp.ones((16,), F32)

    @pl.when(base < NCHUNK)
    def _():
        pltpu.async_copy(xw_hbm.at[idxn.at[0]], xbuf0, sem0)

    def emit(i, tt, xbuf, sem):
        for j in range(8):
            hi = idxh[i, pl.ds(j * 16, 16)]
            v0 = plsc.load_gather(mark0_v, [hi])
            v1 = plsc.load_gather(mark1_v, [hi])
            rows = iota + (j * 16)
            plsc.store_scatter(mbuf, [rows, col0], v0)
            plsc.store_scatter(mbuf, [rows, col1], v1)
            ni = idxn[i, pl.ds(j * 16, 16)]
            plsc.addupdate_scatter(cnt_v, [ni], ones_f)
        pltpu.make_async_copy(xw_hbm.at[idxn.at[i]], xbuf, sem).wait()
        pltpu.sync_copy(xbuf, g1_hbm.at[pl.ds(tt * 128, 128)])
        pltpu.sync_copy(mbuf, m16_hbm.at[pl.ds(tt * 128, 128)])

    def body(g, carry):
        i0 = 2 * g
        i1 = i0 + 1
        t0 = base + i0
        t1 = base + i1

        @pl.when(t1 < NCHUNK)
        def _():
            pltpu.async_copy(xw_hbm.at[idxn.at[i1]], xbuf1, sem1)

        @pl.when(t0 < NCHUNK)
        def _():
            emit(i0, t0, xbuf0, sem0)

        @pl.when(((i0 + 2) < CPT) & ((t0 + 2) < NCHUNK))
        def _():
            pltpu.async_copy(xw_hbm.at[idxn.at[i0 + 2]], xbuf0, sem0)

        @pl.when(t1 < NCHUNK)
        def _():
            emit(i1, t1, xbuf1, sem1)

        return carry

    lax.fori_loop(0, CPT // 2, body, 0)
    pltpu.sync_copy(cnt_v, deg_hbm.at[wid])


def _sc_scatter_he_body(NCHUNK, CPT, SPR,
                        msg_hbm, hidx2d_hbm, zeros_hbm, agg_hbm,
                        idxh, rowbuf0, rowbuf1, agg_sp, sem0, sem1):
    c = lax.axis_index("c")
    s = lax.axis_index("s")
    base = s * CPT
    pltpu.sync_copy(zeros_hbm.at[pl.ds(s * SPR, SPR)],
                    agg_sp.at[pl.ds(s * SPR, SPR)])
    plsc.subcore_barrier()

    @pl.when(base < NCHUNK)
    def _():
        pltpu.async_copy(msg_hbm.at[c].at[pl.ds(base * 128, 128)], rowbuf0,
                         sem0)

    def emit(i, tt, rowbuf, sem):
        pltpu.make_async_copy(msg_hbm.at[c].at[pl.ds(tt * 128, 128)], rowbuf,
                              sem).wait()
        pltpu.sync_copy(rowbuf, agg_sp.at[idxh.at[i]], add=True)

    def body(g, carry):
        i0 = 2 * g
        i1 = i0 + 1
        t0 = base + i0
        t1 = base + i1

        @pl.when(t1 < NCHUNK)
        def _():
            pltpu.async_copy(msg_hbm.at[c].at[pl.ds(t1 * 128, 128)], rowbuf1,
                             sem1)

        pltpu.sync_copy(hidx2d_hbm.at[pl.ds(t0, 2)], idxh)

        @pl.when(t0 < NCHUNK)
        def _():
            emit(0, t0, rowbuf0, sem0)

        @pl.when(((i0 + 2) < CPT) & ((t0 + 2) < NCHUNK))
        def _():
            pltpu.async_copy(msg_hbm.at[c].at[pl.ds((t0 + 2) * 128, 128)],
                             rowbuf0, sem0)

        @pl.when(t1 < NCHUNK)
        def _():
            emit(1, t1, rowbuf1, sem1)

        return carry

    lax.fori_loop(0, CPT // 2, body, 0)
    plsc.subcore_barrier()
    pltpu.sync_copy(agg_sp.at[pl.ds(s * SPR, SPR)],
                    agg_hbm.at[c].at[pl.ds(s * SPR, SPR)])


def _sc_scatter_node_body(NCHUNK, CPT, SPR,
                          mh_hbm, hidx2d_hbm, nidx2d_hbm, zeros_hbm,
                          out_hbm,
                          idxh4, idxn4, rowbuf0, rowbuf1, out_sp,
                          sem0, sem1):
    c = lax.axis_index("c")
    s = lax.axis_index("s")
    base = s * CPT
    pltpu.sync_copy(zeros_hbm.at[pl.ds(s * SPR, SPR)],
                    out_sp.at[pl.ds(s * SPR, SPR)])
    plsc.subcore_barrier()
    pltpu.sync_copy(hidx2d_hbm.at[pl.ds(base, 2)], idxh4.at[pl.ds(0, 2)])
    pltpu.sync_copy(nidx2d_hbm.at[pl.ds(base, 2)], idxn4.at[pl.ds(0, 2)])

    @pl.when(base < NCHUNK)
    def _():
        pltpu.async_copy(mh_hbm.at[c].at[idxh4.at[0]], rowbuf0, sem0)

    def body(g, carry):
        i0 = 2 * g
        t0 = base + i0
        t1 = t0 + 1
        bank = (g % 2) * 2
        nbank = 2 - bank

        @pl.when(t1 < NCHUNK)
        def _():
            pltpu.async_copy(mh_hbm.at[c].at[idxh4.at[bank + 1]], rowbuf1,
                             sem1)

        pltpu.sync_copy(hidx2d_hbm.at[pl.ds(t0 + 2, 2)],
                        idxh4.at[pl.ds(nbank, 2)])
        pltpu.sync_copy(nidx2d_hbm.at[pl.ds(t0 + 2, 2)],
                        idxn4.at[pl.ds(nbank, 2)])

        @pl.when(t0 < NCHUNK)
        def _():
            pltpu.make_async_copy(mh_hbm.at[c].at[idxh4.at[bank]], rowbuf0,
                                  sem0).wait()
            pltpu.sync_copy(rowbuf0, out_sp.at[idxn4.at[bank]], add=True)

        @pl.when(((i0 + 2) < CPT) & ((t0 + 2) < NCHUNK))
        def _():
            pltpu.async_copy(mh_hbm.at[c].at[idxh4.at[nbank]], rowbuf0, sem0)

        @pl.when(t1 < NCHUNK)
        def _():
            pltpu.make_async_copy(mh_hbm.at[c].at[idxh4.at[bank + 1]],
                                  rowbuf1, sem1).wait()
            pltpu.sync_copy(rowbuf1, out_sp.at[idxn4.at[bank + 1]], add=True)

        return carry

    lax.fori_loop(0, CPT // 2, body, 0)
    plsc.subcore_barrier()
    pltpu.sync_copy(out_sp.at[pl.ds(s * SPR, SPR)],
                    out_hbm.at[c].at[pl.ds(s * SPR, SPR)])


def kernel(x, he_index, he_attr, he_mark, he_count,
           n2e_W1, n2e_b1, n2e_g1, n2e_be1, n2e_W2, n2e_b2,
           e2n_W1, e2n_b1, e2n_g1, e2n_be1, e2n_W2, e2n_b2,
           ln_g, ln_b):
    N, D = x.shape
    H, HE = he_attr.shape
    E = he_index.shape[1]
    HID = n2e_W1.shape[1]
    DH = D // 2
    NCHUNK = E // 128
    HP = 10240
    SPR = HP // 16

    node_ids = he_index[0]
    he_ids = he_index[1]
    NCP = 1280
    CPT1 = NCP // 32
    CPT2 = NCP // 16
    hidx2d = jnp.pad(he_ids.reshape(NCHUNK, 128),
                     ((0, NCP + 32 - NCHUNK), (0, 0)))
    nidx2d = jnp.pad(node_ids.reshape(NCHUNK, 128),
                     ((0, NCP + 32 - NCHUNK), (0, 0)))

    mark_c0 = he_mark[:, 0]
    mark_c1 = he_mark[:, 1]
    W1b16 = jnp.pad(n2e_W1[D:], ((0, 16 - (n2e_W1.shape[0] - D)), (0, 0)))
    W1a = n2e_W1[:D]
    V1a = e2n_W1[:HE]
    V1b = e2n_W1[HE:]
    b1 = n2e_b1.reshape(1, HID)
    g1 = n2e_g1.reshape(1, HID)
    be1 = n2e_be1.reshape(1, HID)
    b2 = n2e_b2.reshape(1, HID)
    eb1 = e2n_b1.reshape(1, HID)
    eg1 = e2n_g1.reshape(1, HID)
    ebe1 = e2n_be1.reshape(1, HID)
    eb2 = e2n_b2.reshape(1, D)
    cnt = he_count.reshape(H, 1)
    lng = ln_g.reshape(1, D)
    lnb = ln_b.reshape(1, D)

    zeros_half = jnp.zeros((HP, DH), F32)
    zeros_1d = jnp.zeros((HP,), F32)
    zeros_m = jnp.zeros((128, 16), F32)

    BN = 2000
    xW = pl.pallas_call(
        _tc1_body,
        grid=(N // BN,),
        in_specs=[pl.BlockSpec((BN, D), lambda i: (i, 0)),
                  pl.BlockSpec((D, HID), lambda i: (0, 0))],
        out_specs=pl.BlockSpec((BN, HID), lambda i: (i, 0)),
        out_shape=jax.ShapeDtypeStruct((N, HID), F32),
    )(x, W1a)

    mesh = plsc.VectorSubcoreMesh(**_MESH)
    sc1 = functools.partial(
        pl.kernel,
        out_type=[jax.ShapeDtypeStruct((E, HID), F32),
                  jax.ShapeDtypeStruct((E, 16), F32),
                  jax.ShapeDtypeStruct((32, HP), F32)],
        mesh=mesh,
        compiler_params=pltpu.CompilerParams(needs_layout_passes=False),
        scratch_types=[
            pltpu.VMEM((CPT1, 128), jnp.int32),
            pltpu.VMEM((CPT1, 128), jnp.int32),
            pltpu.VMEM((128, HID), F32),
            pltpu.VMEM((128, HID), F32),
            pltpu.VMEM((128, 16), F32),
            pltpu.VMEM((H,), F32),
            pltpu.VMEM((H,), F32),
            pltpu.VMEM((HP,), F32),
            pltpu.SemaphoreType.DMA,
            pltpu.SemaphoreType.DMA,
        ],
    )(functools.partial(_sc_gather_body, NCHUNK, CPT1))
    G1, M16, deg32 = sc1(xW, mark_c0, mark_c1, nidx2d, hidx2d, zeros_m,
                         zeros_1d)
    deg32 = deg32.T

    BE = 3200
    msg2 = pl.pallas_call(
        _tc2_body,
        grid=(E // BE,),
        in_specs=[pl.BlockSpec((BE, HID), lambda i: (i, 0)),
                  pl.BlockSpec((BE, 16), lambda i: (i, 0)),
                  pl.BlockSpec((16, HID), lambda i: (0, 0)),
                  pl.BlockSpec((1, HID), lambda i: (0, 0)),
                  pl.BlockSpec((1, HID), lambda i: (0, 0)),
                  pl.BlockSpec((1, HID), lambda i: (0, 0)),
                  pl.BlockSpec((HID, HID), lambda i: (0, 0)),
                  pl.BlockSpec((1, HID), lambda i: (0, 0))],
        out_specs=pl.BlockSpec((2, BE, DH), lambda i: (0, i, 0)),
        out_shape=jax.ShapeDtypeStruct((2, E, DH), F32),
    )(G1, M16, W1b16, b1, g1, be1, n2e_W2, b2)

    sc2 = functools.partial(
        pl.kernel,
        out_type=jax.ShapeDtypeStruct((2, HP, DH), F32),
        mesh=mesh,
        compiler_params=pltpu.CompilerParams(needs_layout_passes=False),
        scratch_types=[
            pltpu.VMEM((2, 128), jnp.int32),
            pltpu.VMEM((128, DH), F32),
            pltpu.VMEM((128, DH), F32),
            pltpu.VMEM_SHARED((HP, DH), F32),
            pltpu.SemaphoreType.DMA,
            pltpu.SemaphoreType.DMA,
        ],
    )(functools.partial(_sc_scatter_he_body, NCHUNK, CPT2, SPR))
    agg2 = sc2(msg2, hidx2d, zeros_half)

    BH = 2000
    mh2 = pl.pallas_call(
        _tc3_body,
        grid=(H // BH,),
        in_specs=[pl.BlockSpec((2, BH, DH), lambda i: (0, i, 0)),
                  pl.BlockSpec((BH, HE), lambda i: (i, 0)),
                  pl.BlockSpec((BH, 1), lambda i: (i, 0)),
                  pl.BlockSpec((HE, HID), lambda i: (0, 0)),
                  pl.BlockSpec((HID, HID), lambda i: (0, 0)),
                  pl.BlockSpec((1, HID), lambda i: (0, 0)),
                  pl.BlockSpec((1, HID), lambda i: (0, 0)),
                  pl.BlockSpec((1, HID), lambda i: (0, 0)),
                  pl.BlockSpec((HID, D), lambda i: (0, 0)),
                  pl.BlockSpec((1, D), lambda i: (0, 0))],
        out_specs=pl.BlockSpec((2, BH, DH), lambda i: (0, i, 0)),
        out_shape=jax.ShapeDtypeStruct((2, HP, DH), F32),
    )(agg2, he_attr, cnt, V1a, V1b, eb1, eg1, ebe1, e2n_W2, eb2)

    sc3 = functools.partial(
        pl.kernel,
        out_type=jax.ShapeDtypeStruct((2, HP, DH), F32),
        mesh=mesh,
        compiler_params=pltpu.CompilerParams(needs_layout_passes=False),
        scratch_types=[
            pltpu.VMEM((4, 128), jnp.int32),
            pltpu.VMEM((4, 128), jnp.int32),
            pltpu.VMEM((128, DH), F32),
            pltpu.VMEM((128, DH), F32),
            pltpu.VMEM_SHARED((HP, DH), F32),
            pltpu.SemaphoreType.DMA,
            pltpu.SemaphoreType.DMA,
        ],
    )(functools.partial(_sc_scatter_node_body, NCHUNK, CPT2, SPR))
    out2 = sc3(mh2, hidx2d, nidx2d, zeros_half)

    y = pl.pallas_call(
        _tc4_body,
        grid=(N // BN,),
        in_specs=[pl.BlockSpec((2, BN, DH), lambda i: (0, i, 0)),
                  pl.BlockSpec((BN, 32), lambda i: (i, 0)),
                  pl.BlockSpec((BN, D), lambda i: (i, 0)),
                  pl.BlockSpec((1, D), lambda i: (0, 0)),
                  pl.BlockSpec((1, D), lambda i: (0, 0))],
        out_specs=pl.BlockSpec((BN, D), lambda i: (i, 0)),
        out_shape=jax.ShapeDtypeStruct((N, D), F32),
    )(out2, deg32, x, lng, lnb)
    return y


# speedup vs baseline: 5.4608x; 1.0351x over previous
import functools

import jax
import jax.numpy as jnp
from jax import lax
from jax.experimental import pallas as pl
from jax.experimental.pallas import tpu as pltpu
from jax.experimental.pallas import tpu_sc as plsc

F32 = jnp.float32


def _tc1_body(x_ref, w_ref, o_ref):
    o_ref[...] = jnp.dot(x_ref[...], w_ref[...], preferred_element_type=F32)


def _layernorm_rows(h, g, b):
    mu = jnp.mean(h, axis=1, keepdims=True)
    var = jnp.mean((h - mu) * (h - mu), axis=1, keepdims=True)
    return (h - mu) / jnp.sqrt(var + 1e-5) * g + b


def _tc2_body(g1_ref, m16_ref, w1b_ref, b1_ref, g_ref, be_ref, w2_ref, b2_ref,
              o_ref):
    h = (g1_ref[...]
         + jnp.dot(m16_ref[...], w1b_ref[...], preferred_element_type=F32)
         + b1_ref[...])
    a = jnp.maximum(_layernorm_rows(h, g_ref[...], be_ref[...]), 0.0)
    msg = jnp.dot(a.astype(jnp.bfloat16), w2_ref[...].astype(jnp.bfloat16),
                  preferred_element_type=F32) + b2_ref[...]
    half = msg.shape[1] // 2
    o_ref[0, ...] = msg[:, :half]
    o_ref[1, ...] = msg[:, half:]


def _tc3_body(agga_ref, aggb_ref, attr_ref, cnt_ref, v1a_ref, v1b_ref,
              b1_ref, g_ref, be_ref, v2_ref, b2_ref, o_ref):
    agg = jnp.concatenate([agga_ref[0, ...] + aggb_ref[0, ...],
                           agga_ref[1, ...] + aggb_ref[1, ...]], axis=1)
    agg = agg / (cnt_ref[...] + 1e-6)
    h = (jnp.dot(attr_ref[...], v1a_ref[...], preferred_element_type=F32)
         + jnp.dot(agg.astype(jnp.bfloat16), v1b_ref[...].astype(jnp.bfloat16),
                   preferred_element_type=F32)
         + b1_ref[...])
    a = jnp.maximum(_layernorm_rows(h, g_ref[...], be_ref[...]), 0.0)
    mh = jnp.maximum(
        jnp.dot(a.astype(jnp.bfloat16), v2_ref[...].astype(jnp.bfloat16),
                preferred_element_type=F32) + b2_ref[...], 0.0)
    half = mh.shape[1] // 2
    o_ref[0, ...] = mh[:, :half]
    o_ref[1, ...] = mh[:, half:]


def _tc4_body(o2_ref, deg_ref, x_ref, g_ref, b_ref, y_ref):
    o = jnp.concatenate([o2_ref[0, ...], o2_ref[1, ...]], axis=1)
    deg = jnp.sum(deg_ref[...], axis=1)[:, None]
    o = o / (deg + 1e-6)
    y_ref[...] = x_ref[...] + _layernorm_rows(o, g_ref[...], b_ref[...])


_MESH = dict(core_axis_name="c", subcore_axis_name="s")


def _sc_gather_body(LO, HI, CPT, xw_hbm, mc0_hbm, mc1_hbm, nidx1d_hbm,
                    hidx1d_hbm, zm_hbm, z1d_hbm, g1_hbm, m16_hbm, deg_hbm,
                    idxn, idxh, xbuf0, xbuf1, mbuf, mark0_v, mark1_v, cnt_v,
                    sem0, sem1):
    c = lax.axis_index("c")
    s = lax.axis_index("s")
    wid = s * 2 + c
    base = LO + wid * CPT
    pltpu.sync_copy(nidx1d_hbm.at[pl.ds(base * 128, CPT * 128)], idxn)
    pltpu.sync_copy(hidx1d_hbm.at[pl.ds(base * 128, CPT * 128)], idxh)
    pltpu.sync_copy(mc0_hbm, mark0_v)
    pltpu.sync_copy(mc1_hbm, mark1_v)
    pltpu.sync_copy(zm_hbm, mbuf)
    pltpu.sync_copy(z1d_hbm, cnt_v)
    iota = lax.iota(jnp.int32, 16)
    col0 = jnp.zeros((16,), jnp.int32)
    col1 = jnp.ones((16,), jnp.int32)
    ones_f = jnp.ones((16,), F32)

    @pl.when(base < HI)
    def _():
        pltpu.async_copy(xw_hbm.at[idxn.at[pl.ds(0, 128)]], xbuf0, sem0)

    def emit(i, tt, xbuf, sem):
        for j in range(8):
            hi = idxh[pl.ds(i * 128 + j * 16, 16)]
            v0 = plsc.load_gather(mark0_v, [hi])
            v1 = plsc.load_gather(mark1_v, [hi])
            rows = iota + (j * 16)
            plsc.store_scatter(mbuf, [rows, col0], v0)
            plsc.store_scatter(mbuf, [rows, col1], v1)
            ni = idxn[pl.ds(i * 128 + j * 16, 16)]
            plsc.addupdate_scatter(cnt_v, [ni], ones_f)
        pltpu.make_async_copy(xw_hbm.at[idxn.at[pl.ds(i * 128, 128)]], xbuf,
                              sem).wait()
        pltpu.sync_copy(xbuf, g1_hbm.at[pl.ds((tt - LO) * 128, 128)])
        pltpu.sync_copy(mbuf, m16_hbm.at[pl.ds((tt - LO) * 128, 128)])

    def body(g, carry):
        i0 = 2 * g
        i1 = i0 + 1
        t0 = base + i0
        t1 = base + i1

        @pl.when(t1 < HI)
        def _():
            pltpu.async_copy(xw_hbm.at[idxn.at[pl.ds(i1 * 128, 128)]], xbuf1,
                             sem1)

        @pl.when(t0 < HI)
        def _():
            emit(i0, t0, xbuf0, sem0)

        @pl.when(((i0 + 2) < CPT) & ((t0 + 2) < HI))
        def _():
            pltpu.async_copy(xw_hbm.at[idxn.at[pl.ds((i0 + 2) * 128, 128)]],
                             xbuf0, sem0)

        @pl.when(t1 < HI)
        def _():
            emit(i1, t1, xbuf1, sem1)

        return carry

    lax.fori_loop(0, CPT // 2, body, 0)
    pltpu.sync_copy(cnt_v, deg_hbm.at[wid])


def _sc_scatter_he_body(LO, HI, CPT, SPR,
                        msg_hbm, hidx2d_hbm, zeros_hbm, agg_hbm,
                        idxh, rowbuf0, rowbuf1, agg_sp, sem0, sem1):
    c = lax.axis_index("c")
    s = lax.axis_index("s")
    base = LO + s * CPT
    pltpu.sync_copy(zeros_hbm.at[pl.ds(s * SPR, SPR)],
                    agg_sp.at[pl.ds(s * SPR, SPR)])
    plsc.subcore_barrier()

    @pl.when(base < HI)
    def _():
        pltpu.async_copy(msg_hbm.at[c].at[pl.ds((base - LO) * 128, 128)],
                         rowbuf0, sem0)

    def emit(i, tt, rowbuf, sem):
        pltpu.make_async_copy(msg_hbm.at[c].at[pl.ds((tt - LO) * 128, 128)],
                              rowbuf, sem).wait()
        pltpu.sync_copy(rowbuf, agg_sp.at[idxh.at[i]], add=True)

    def body(g, carry):
        i0 = 2 * g
        i1 = i0 + 1
        t0 = base + i0
        t1 = base + i1

        @pl.when(t1 < HI)
        def _():
            pltpu.async_copy(msg_hbm.at[c].at[pl.ds((t1 - LO) * 128, 128)],
                             rowbuf1, sem1)

        pltpu.sync_copy(hidx2d_hbm.at[pl.ds(t0, 2)], idxh)

        @pl.when(t0 < HI)
        def _():
            emit(0, t0, rowbuf0, sem0)

        @pl.when(((i0 + 2) < CPT) & ((t0 + 2) < HI))
        def _():
            pltpu.async_copy(msg_hbm.at[c].at[pl.ds((t0 + 2 - LO) * 128, 128)],
                             rowbuf0, sem0)

        @pl.when(t1 < HI)
        def _():
            emit(1, t1, rowbuf1, sem1)

        return carry

    lax.fori_loop(0, CPT // 2, body, 0)
    plsc.subcore_barrier()
    pltpu.sync_copy(agg_sp.at[pl.ds(s * SPR, SPR)],
                    agg_hbm.at[c].at[pl.ds(s * SPR, SPR)])


def _sc_scatter_node_body(NCHUNK, CPT, SPR,
                          mh_hbm, hidx2d_hbm, nidx2d_hbm, zeros_hbm,
                          out_hbm,
                          idxh4, idxn4, rowbuf0, rowbuf1, out_sp,
                          sem0, sem1):
    c = lax.axis_index("c")
    s = lax.axis_index("s")
    base = s * CPT
    pltpu.sync_copy(zeros_hbm.at[pl.ds(s * SPR, SPR)],
                    out_sp.at[pl.ds(s * SPR, SPR)])
    plsc.subcore_barrier()
    pltpu.sync_copy(hidx2d_hbm.at[pl.ds(base, 2)], idxh4.at[pl.ds(0, 2)])
    pltpu.sync_copy(nidx2d_hbm.at[pl.ds(base, 2)], idxn4.at[pl.ds(0, 2)])

    @pl.when(base < NCHUNK)
    def _():
        pltpu.async_copy(mh_hbm.at[c].at[idxh4.at[0]], rowbuf0, sem0)

    def body(g, carry):
        i0 = 2 * g
        t0 = base + i0
        t1 = t0 + 1
        bank = (g % 2) * 2
        nbank = 2 - bank

        @pl.when(t1 < NCHUNK)
        def _():
            pltpu.async_copy(mh_hbm.at[c].at[idxh4.at[bank + 1]], rowbuf1,
                             sem1)

        pltpu.sync_copy(hidx2d_hbm.at[pl.ds(t0 + 2, 2)],
                        idxh4.at[pl.ds(nbank, 2)])
        pltpu.sync_copy(nidx2d_hbm.at[pl.ds(t0 + 2, 2)],
                        idxn4.at[pl.ds(nbank, 2)])

        @pl.when(t0 < NCHUNK)
        def _():
            pltpu.make_async_copy(mh_hbm.at[c].at[idxh4.at[bank]], rowbuf0,
                                  sem0).wait()
            pltpu.sync_copy(rowbuf0, out_sp.at[idxn4.at[bank]], add=True)

        @pl.when(((i0 + 2) < CPT) & ((t0 + 2) < NCHUNK))
        def _():
            pltpu.async_copy(mh_hbm.at[c].at[idxh4.at[nbank]], rowbuf0, sem0)

        @pl.when(t1 < NCHUNK)
        def _():
            pltpu.make_async_copy(mh_hbm.at[c].at[idxh4.at[bank + 1]],
                                  rowbuf1, sem1).wait()
            pltpu.sync_copy(rowbuf1, out_sp.at[idxn4.at[bank + 1]], add=True)

        return carry

    lax.fori_loop(0, CPT // 2, body, 0)
    plsc.subcore_barrier()
    pltpu.sync_copy(out_sp.at[pl.ds(s * SPR, SPR)],
                    out_hbm.at[c].at[pl.ds(s * SPR, SPR)])


def kernel(x, he_index, he_attr, he_mark, he_count,
           n2e_W1, n2e_b1, n2e_g1, n2e_be1, n2e_W2, n2e_b2,
           e2n_W1, e2n_b1, e2n_g1, e2n_be1, e2n_W2, e2n_b2,
           ln_g, ln_b):
    N, D = x.shape
    H, HE = he_attr.shape
    E = he_index.shape[1]
    HID = n2e_W1.shape[1]
    DH = D // 2
    NCHUNK = E // 128
    HP = 10240
    SPR = HP // 16

    node_ids = he_index[0]
    he_ids = he_index[1]
    NCP = 1280
    CPT1 = NCP // 32
    CPT2 = NCP // 16
    hidx2d = jnp.pad(he_ids.reshape(NCHUNK, 128),
                     ((0, NCP + 32 - NCHUNK), (0, 0)))
    nidx2d = jnp.pad(node_ids.reshape(NCHUNK, 128),
                     ((0, NCP + 32 - NCHUNK), (0, 0)))

    mark_c0 = he_mark[:, 0]
    mark_c1 = he_mark[:, 1]
    W1b16 = jnp.pad(n2e_W1[D:], ((0, 16 - (n2e_W1.shape[0] - D)), (0, 0)))
    W1a = n2e_W1[:D]
    V1a = e2n_W1[:HE]
    V1b = e2n_W1[HE:]
    b1 = n2e_b1.reshape(1, HID)
    g1 = n2e_g1.reshape(1, HID)
    be1 = n2e_be1.reshape(1, HID)
    b2 = n2e_b2.reshape(1, HID)
    eb1 = e2n_b1.reshape(1, HID)
    eg1 = e2n_g1.reshape(1, HID)
    ebe1 = e2n_be1.reshape(1, HID)
    eb2 = e2n_b2.reshape(1, D)
    cnt = he_count.reshape(H, 1)
    lng = ln_g.reshape(1, D)
    lnb = ln_b.reshape(1, D)

    zeros_half = jnp.zeros((HP, DH), F32)
    zeros_1d = jnp.zeros((HP,), F32)
    zeros_m = jnp.zeros((128, 16), F32)

    BN = 2000
    xW = pl.pallas_call(
        _tc1_body,
        grid=(N // BN,),
        in_specs=[pl.BlockSpec((BN, D), lambda i: (i, 0)),
                  pl.BlockSpec((D, HID), lambda i: (0, 0))],
        out_specs=pl.BlockSpec((BN, HID), lambda i: (i, 0)),
        out_shape=jax.ShapeDtypeStruct((N, HID), F32),
    )(x, W1a)

    mesh = plsc.VectorSubcoreMesh(**_MESH)
    CH = 624
    EHA = CH * 128
    EHB = (NCHUNK - CH) * 128
    CPT1H = 20
    CPT2H = 640 // 16

    def make_sc1(lo, hi, eh):
        return functools.partial(
            pl.kernel,
            out_type=[jax.ShapeDtypeStruct((eh, HID), F32),
                      jax.ShapeDtypeStruct((eh, 16), F32),
                      jax.ShapeDtypeStruct((32, HP), F32)],
            mesh=mesh,
            compiler_params=pltpu.CompilerParams(needs_layout_passes=False),
            scratch_types=[
                pltpu.VMEM((CPT1H * 128,), jnp.int32),
                pltpu.VMEM((CPT1H * 128,), jnp.int32),
                pltpu.VMEM((128, HID), F32),
                pltpu.VMEM((128, HID), F32),
                pltpu.VMEM((128, 16), F32),
                pltpu.VMEM((H,), F32),
                pltpu.VMEM((H,), F32),
                pltpu.VMEM((HP,), F32),
                pltpu.SemaphoreType.DMA,
                pltpu.SemaphoreType.DMA,
            ],
        )(functools.partial(_sc_gather_body, lo, hi, CPT1H))

    nidx1d = nidx2d.reshape(-1)
    hidx1d = hidx2d.reshape(-1)
    G1a, M16a, deg32a = make_sc1(0, CH, EHA)(xW, mark_c0, mark_c1, nidx1d,
                                             hidx1d, zeros_m, zeros_1d)
    G1b, M16b, deg32b = make_sc1(CH, NCHUNK, EHB)(xW, mark_c0, mark_c1,
                                                  nidx1d, hidx1d, zeros_m,
                                                  zeros_1d)
    deg64 = jnp.concatenate([deg32a, deg32b], axis=0).T

    BE = 3328

    def run_tc2(g1h, m16h):
        eh = g1h.shape[0]
        return pl.pallas_call(
            _tc2_body,
            grid=(pl.cdiv(eh, BE),),
            in_specs=[pl.BlockSpec((BE, HID), lambda i: (i, 0)),
                      pl.BlockSpec((BE, 16), lambda i: (i, 0)),
                      pl.BlockSpec((16, HID), lambda i: (0, 0)),
                      pl.BlockSpec((1, HID), lambda i: (0, 0)),
                      pl.BlockSpec((1, HID), lambda i: (0, 0)),
                      pl.BlockSpec((1, HID), lambda i: (0, 0)),
                      pl.BlockSpec((HID, HID), lambda i: (0, 0)),
                      pl.BlockSpec((1, HID), lambda i: (0, 0))],
            out_specs=pl.BlockSpec((2, BE, DH), lambda i: (0, i, 0)),
            out_shape=jax.ShapeDtypeStruct((2, eh, DH), F32),
        )(g1h, m16h, W1b16, b1, g1, be1, n2e_W2, b2)

    msg2a = run_tc2(G1a, M16a)
    msg2b = run_tc2(G1b, M16b)

    def make_sc2(lo, hi):
        return functools.partial(
            pl.kernel,
            out_type=jax.ShapeDtypeStruct((2, HP, DH), F32),
            mesh=mesh,
            compiler_params=pltpu.CompilerParams(needs_layout_passes=False),
            scratch_types=[
                pltpu.VMEM((2, 128), jnp.int32),
                pltpu.VMEM((128, DH), F32),
                pltpu.VMEM((128, DH), F32),
                pltpu.VMEM_SHARED((HP, DH), F32),
                pltpu.SemaphoreType.DMA,
                pltpu.SemaphoreType.DMA,
            ],
        )(functools.partial(_sc_scatter_he_body, lo, hi, CPT2H, SPR))

    agg2a = make_sc2(0, CH)(msg2a, hidx2d, zeros_half)
    agg2b = make_sc2(CH, NCHUNK)(msg2b, hidx2d, zeros_half)

    BH = 2000
    mh2 = pl.pallas_call(
        _tc3_body,
        grid=(H // BH,),
        in_specs=[pl.BlockSpec((2, BH, DH), lambda i: (0, i, 0)),
                  pl.BlockSpec((2, BH, DH), lambda i: (0, i, 0)),
                  pl.BlockSpec((BH, HE), lambda i: (i, 0)),
                  pl.BlockSpec((BH, 1), lambda i: (i, 0)),
                  pl.BlockSpec((HE, HID), lambda i: (0, 0)),
                  pl.BlockSpec((HID, HID), lambda i: (0, 0)),
                  pl.BlockSpec((1, HID), lambda i: (0, 0)),
                  pl.BlockSpec((1, HID), lambda i: (0, 0)),
                  pl.BlockSpec((1, HID), lambda i: (0, 0)),
                  pl.BlockSpec((HID, D), lambda i: (0, 0)),
                  pl.BlockSpec((1, D), lambda i: (0, 0))],
        out_specs=pl.BlockSpec((2, BH, DH), lambda i: (0, i, 0)),
        out_shape=jax.ShapeDtypeStruct((2, HP, DH), F32),
    )(agg2a, agg2b, he_attr, cnt, V1a, V1b, eb1, eg1, ebe1, e2n_W2, eb2)

    sc3 = functools.partial(
        pl.kernel,
        out_type=jax.ShapeDtypeStruct((2, HP, DH), F32),
        mesh=mesh,
        compiler_params=pltpu.CompilerParams(needs_layout_passes=False),
        scratch_types=[
            pltpu.VMEM((4, 128), jnp.int32),
            pltpu.VMEM((4, 128), jnp.int32),
            pltpu.VMEM((128, DH), F32),
            pltpu.VMEM((128, DH), F32),
            pltpu.VMEM_SHARED((HP, DH), F32),
            pltpu.SemaphoreType.DMA,
            pltpu.SemaphoreType.DMA,
        ],
    )(functools.partial(_sc_scatter_node_body, NCHUNK, CPT2, SPR))
    out2 = sc3(mh2, hidx2d, nidx2d, zeros_half)

    y = pl.pallas_call(
        _tc4_body,
        grid=(N // BN,),
        in_specs=[pl.BlockSpec((2, BN, DH), lambda i: (0, i, 0)),
                  pl.BlockSpec((BN, 64), lambda i: (i, 0)),
                  pl.BlockSpec((BN, D), lambda i: (i, 0)),
                  pl.BlockSpec((1, D), lambda i: (0, 0)),
                  pl.BlockSpec((1, D), lambda i: (0, 0))],
        out_specs=pl.BlockSpec((BN, D), lambda i: (i, 0)),
        out_shape=jax.ShapeDtypeStruct((N, D), F32),
    )(out2, deg64, x, lng, lnb)
    return y
